# Initial kernel scaffold; baseline (speedup 1.0000x reference)
import jax
import jax.numpy as jnp
from jax.experimental import pallas as pl


def kernel(x, gating_output, w13, w2):
    raise NotImplementedError("write your pallas kernel here")


# speedup vs baseline: 1.4833x; 1.4833x over previous
import functools

import jax
import jax.numpy as jnp
from jax.experimental import pallas as pl

E = 8
K = 2
FCHUNK = 512


def _routing_weights(gating, e):
    t, n = gating.shape
    m = jnp.max(gating, axis=1, keepdims=True)
    p = jnp.exp(gating - m)
    rw = p / jnp.sum(p, axis=1, keepdims=True)
    colid = jax.lax.broadcasted_iota(jnp.int32, rw.shape, 1)
    m1 = jnp.max(rw, axis=1, keepdims=True)
    i1 = jnp.min(jnp.where(rw == m1, colid, n), axis=1, keepdims=True)
    is1 = colid == i1
    rw_m = jnp.where(is1, -jnp.inf, rw)
    m2 = jnp.max(rw_m, axis=1, keepdims=True)
    i2 = jnp.min(jnp.where(rw_m == m2, colid, n), axis=1, keepdims=True)
    sel = is1 | (colid == i2)
    wmat = jnp.where(sel, rw, 0.0) / (m1 + m2)
    return jnp.sum(jnp.where(colid == e, wmat, 0.0), axis=1, keepdims=True)


def _moe_body(x_ref, gating_ref, w13g_ref, w13u_ref, w2_ref, out_ref):
    e = pl.program_id(0)
    f = pl.program_id(1)
    wcol = _routing_weights(gating_ref[...], e)

    xt = x_ref[...]
    g = jax.lax.dot_general(xt, w13g_ref[0],
                            (((1,), (1,)), ((), ())),
                            preferred_element_type=jnp.float32)
    u = jax.lax.dot_general(xt, w13u_ref[0],
                            (((1,), (1,)), ((), ())),
                            preferred_element_type=jnp.float32)
    h = g * jax.nn.sigmoid(g) * u
    y = jax.lax.dot_general(h, w2_ref[0],
                            (((1,), (1,)), ((), ())),
                            preferred_element_type=jnp.float32)
    contrib = y * wcol

    @pl.when((e == 0) & (f == 0))
    def _():
        out_ref[...] = contrib

    @pl.when((e > 0) | (f > 0))
    def _():
        out_ref[...] = out_ref[...] + contrib


@functools.partial(jax.jit, static_argnames=())
def kernel(x, gating_output, w13, w2):
    T, D = x.shape
    F = w2.shape[2]
    nf = F // FCHUNK
    out = pl.pallas_call(
        _moe_body,
        grid=(E, nf),
        in_specs=[
            pl.BlockSpec((T, D), lambda e, f: (0, 0)),
            pl.BlockSpec((T, E), lambda e, f: (0, 0)),
            pl.BlockSpec((1, FCHUNK, D), lambda e, f: (e, f, 0)),
            pl.BlockSpec((1, FCHUNK, D), lambda e, f: (e, nf + f, 0)),
            pl.BlockSpec((1, D, FCHUNK), lambda e, f: (e, 0, f)),
        ],
        out_specs=pl.BlockSpec((T, D), lambda e, f: (0, 0)),
        out_shape=jax.ShapeDtypeStruct((T, D), jnp.float32),
    )(x, gating_output, w13, w13, w2)
    return out
